# baseline (speedup 1.0000x reference)
import functools

import jax
import jax.numpy as jnp
from jax import lax
from jax.experimental import pallas as pl
from jax.experimental.pallas import tpu as pltpu
from jax.experimental.pallas import tpu_sc as plsc

_NUM_WORKERS = 32
_CHUNK = 128


def _sc_gather(ids, E0, E1, E2):
    _, nw, nchunk, chunk = ids.shape
    bpw = nchunk * chunk
    b = nw * bpw
    d = E0.shape[1]
    mesh = plsc.VectorSubcoreMesh(core_axis_name="c", subcore_axis_name="s")

    @functools.partial(
        pl.kernel,
        mesh=mesh,
        compiler_params=pltpu.CompilerParams(use_tc_tiling_on_sc=False),
        out_type=[jax.ShapeDtypeStruct((b, d), jnp.float32)] * 3,
        scratch_types=[
            pltpu.VMEM((nchunk, chunk), jnp.int32),
            pltpu.VMEM((bpw, d), jnp.float32),
            pltpu.SemaphoreType.DMA,
        ],
    )
    def k(ids_hbm, e0, e1, e2, o0, o1, o2, idx_v, rows_v, sem):
        wid = lax.axis_index("s") * 2 + lax.axis_index("c")
        base = wid * bpw
        for t, (e, o) in enumerate(((e0, o0), (e1, o1), (e2, o2))):
            pltpu.sync_copy(ids_hbm.at[t, wid], idx_v)
            copies = []
            for c in range(nchunk):
                copies.append(
                    pltpu.async_copy(
                        e.at[idx_v.at[c]],
                        rows_v.at[pl.ds(c * chunk, chunk)],
                        sem,
                    )
                )
            for cp in copies:
                cp.wait()
            pltpu.sync_copy(rows_v, o.at[pl.ds(base, bpw)])

    return k(ids, E0, E1, E2)


def _tc_fused(cls_token, v0, v1, v2, W1, b1, W2, b2, Wg, bg, block_b=512):
    b, c = cls_token.shape
    d = v0.shape[1]
    t = c + 3 * d

    def body(cls_ref, v0_ref, v1_ref, v2_ref, w1_ref, b1_ref, w2_ref,
             b2_ref, wg_ref, bg_ref, out_ref):
        x = jnp.concatenate(
            [cls_ref[...], v0_ref[...], v1_ref[...], v2_ref[...]], axis=1)
        h = jnp.dot(x, w1_ref[...], preferred_element_type=jnp.float32)
        h = h + b1_ref[...]
        h = 0.5 * h * (1.0 + lax.erf(h * 0.7071067811865476))
        meta = jnp.dot(h, w2_ref[...], preferred_element_type=jnp.float32)
        meta = meta + b2_ref[...]
        g = jnp.dot(x, wg_ref[...], preferred_element_type=jnp.float32)
        gate = jax.nn.sigmoid(g + bg_ref[...])
        out_ref[...] = meta * gate

    const = lambda i: (0, 0)
    batch = lambda i: (i, 0)
    return pl.pallas_call(
        body,
        grid=(b // block_b,),
        in_specs=[
            pl.BlockSpec((block_b, c), batch),
            pl.BlockSpec((block_b, d), batch),
            pl.BlockSpec((block_b, d), batch),
            pl.BlockSpec((block_b, d), batch),
            pl.BlockSpec((t, c), const),
            pl.BlockSpec((1, c), const),
            pl.BlockSpec((c, c), const),
            pl.BlockSpec((1, c), const),
            pl.BlockSpec((t, c), const),
            pl.BlockSpec((1, c), const),
        ],
        out_specs=pl.BlockSpec((block_b, c), batch),
        out_shape=jax.ShapeDtypeStruct((b, c), jnp.float32),
    )(cls_token, v0, v1, v2, W1, b1.reshape(1, c), W2, b2.reshape(1, c),
      Wg, bg.reshape(1, c))


def kernel(cls_token, meta_ids, E0, E1, E2, W1, b1, W2, b2, Wg, bg):
    b = cls_token.shape[0]
    bpw = b // _NUM_WORKERS
    nchunk = bpw // _CHUNK
    ids = meta_ids.astype(jnp.int32).T.reshape(3, _NUM_WORKERS, nchunk, _CHUNK)
    v0, v1, v2 = _sc_gather(ids, E0, E1, E2)
    return _tc_fused(cls_token, v0, v1, v2, W1, b1, W2, b2, Wg, bg)


# speedup vs baseline: 2.3653x; 2.3653x over previous
import functools

import jax
import jax.numpy as jnp
from jax import lax
from jax.experimental import pallas as pl
from jax.experimental.pallas import tpu as pltpu
from jax.experimental.pallas import tpu_sc as plsc

_NUM_WORKERS = 32
_CHUNK = 128
_WAVE = 2


def _tc_split(E, block_rows=2000):
    v, d = E.shape

    def body(e_ref, l_ref, r_ref):
        l_ref[...] = e_ref[:, 0:128]
        tail = e_ref[:, 128:192]
        r_ref[...] = jnp.concatenate([tail, tail], axis=1)

    return pl.pallas_call(
        body,
        grid=(v // block_rows,),
        in_specs=[pl.BlockSpec((block_rows, d), lambda i: (i, 0))],
        out_specs=[pl.BlockSpec((block_rows, 128), lambda i: (i, 0))] * 2,
        out_shape=[jax.ShapeDtypeStruct((v, 128), jnp.float32)] * 2,
    )(E)


def _sc_gather(ids3, L0, R0, L1, R1, L2, R2):
    _, nw, nchunk, chunk = ids3.shape
    bpw = nchunk * chunk
    b = nw * bpw
    wave_rows = _WAVE * chunk
    mesh = plsc.VectorSubcoreMesh(core_axis_name="c", subcore_axis_name="s")

    @functools.partial(
        pl.kernel,
        mesh=mesh,
        out_type=[jax.ShapeDtypeStruct((b, 128), jnp.float32)] * 6,
        scratch_types=[
            pltpu.VMEM((nchunk, chunk), jnp.int32),
            pltpu.VMEM((wave_rows, 128), jnp.float32),
            pltpu.VMEM((wave_rows, 128), jnp.float32),
            pltpu.SemaphoreType.DMA,
        ],
    )
    def k(ids_hbm, l0, r0, l1, r1, l2, r2,
          o0l, o0r, o1l, o1r, o2l, o2r, idx_v, rl_v, rr_v, sem):
        wid = lax.axis_index("s") * 2 + lax.axis_index("c")
        base = wid * bpw
        for t, (l, r, ol, orr) in enumerate(
                ((l0, r0, o0l, o0r), (l1, r1, o1l, o1r), (l2, r2, o2l, o2r))):
            pltpu.sync_copy(ids_hbm.at[t, wid], idx_v)
            for w in range(nchunk // _WAVE):
                copies = []
                for c in range(_WAVE):
                    i_slice = idx_v.at[w * _WAVE + c]
                    dst = pl.ds(c * chunk, chunk)
                    copies.append(pltpu.async_copy(
                        l.at[i_slice], rl_v.at[dst], sem))
                    copies.append(pltpu.async_copy(
                        r.at[i_slice], rr_v.at[dst], sem))
                for cp in copies:
                    cp.wait()
                out_slice = pl.ds(base + w * wave_rows, wave_rows)
                pltpu.sync_copy(rl_v, ol.at[out_slice])
                pltpu.sync_copy(rr_v, orr.at[out_slice])

    return k(ids3, L0, R0, L1, R1, L2, R2)


def _tc_fused(cls_token, parts, W1, b1, W2, b2, Wg, bg, block_b=512):
    b, c = cls_token.shape
    d = 192
    t = c + 3 * d

    def body(cls_ref, p0l_ref, p0r_ref, p1l_ref, p1r_ref, p2l_ref, p2r_ref,
             w1_ref, b1_ref, w2_ref, b2_ref, wg_ref, bg_ref, out_ref):
        x = jnp.concatenate([
            cls_ref[...],
            p0l_ref[...], p0r_ref[:, 0:64],
            p1l_ref[...], p1r_ref[:, 0:64],
            p2l_ref[...], p2r_ref[:, 0:64],
        ], axis=1)
        x = x.astype(jnp.bfloat16)
        h = jnp.dot(x, w1_ref[...], preferred_element_type=jnp.float32)
        h = h + b1_ref[...]
        h = 0.5 * h * (1.0 + lax.erf(h * 0.7071067811865476))
        meta = jnp.dot(h.astype(jnp.bfloat16), w2_ref[...],
                       preferred_element_type=jnp.float32)
        meta = meta + b2_ref[...]
        g = jnp.dot(x, wg_ref[...], preferred_element_type=jnp.float32)
        gate = jax.nn.sigmoid(g + bg_ref[...])
        out_ref[...] = meta * gate

    const = lambda i: (0, 0)
    batch = lambda i: (i, 0)
    return pl.pallas_call(
        body,
        grid=(b // block_b,),
        in_specs=[
            pl.BlockSpec((block_b, c), batch),
        ] + [pl.BlockSpec((block_b, 128), batch)] * 6 + [
            pl.BlockSpec((t, c), const),
            pl.BlockSpec((1, c), const),
            pl.BlockSpec((c, c), const),
            pl.BlockSpec((1, c), const),
            pl.BlockSpec((t, c), const),
            pl.BlockSpec((1, c), const),
        ],
        out_specs=pl.BlockSpec((block_b, c), batch),
        out_shape=jax.ShapeDtypeStruct((b, c), jnp.float32),
    )(cls_token, *parts, W1.astype(jnp.bfloat16), b1.reshape(1, c),
      W2.astype(jnp.bfloat16), b2.reshape(1, c), Wg.astype(jnp.bfloat16),
      bg.reshape(1, c))


def kernel(cls_token, meta_ids, E0, E1, E2, W1, b1, W2, b2, Wg, bg):
    b = cls_token.shape[0]
    bpw = b // _NUM_WORKERS
    nchunk = bpw // _CHUNK

    ids3 = meta_ids.astype(jnp.int32).T.reshape(3, _NUM_WORKERS, nchunk,
                                                _CHUNK)
    L0, R0 = _tc_split(E0)
    L1, R1 = _tc_split(E1)
    L2, R2 = _tc_split(E2)
    parts = _sc_gather(ids3, L0, R0, L1, R1, L2, R2)
    return _tc_fused(cls_token, parts, W1, b1, W2, b2, Wg, bg)


# speedup vs baseline: 2.5975x; 1.0981x over previous
import functools

import jax
import jax.numpy as jnp
from jax import lax
from jax.experimental import pallas as pl
from jax.experimental.pallas import tpu as pltpu
from jax.experimental.pallas import tpu_sc as plsc

_NUM_WORKERS = 32
_CHUNK = 128
_WAVE = 2


def _tc_tail_split(E0, E1, E2, block_rows=2000):
    v = E0.shape[0]

    def body(e0_hbm, e1_hbm, e2_hbm, t01_ref, t2_ref, s0, s1, s2,
             m0, m1, m2):
        i = pl.program_id(0)
        rows = pl.ds(i * block_rows, block_rows)
        tail_cols = pl.ds(128, 64)
        copies = [
            pltpu.make_async_copy(e0_hbm.at[rows, tail_cols], s0, m0),
            pltpu.make_async_copy(e1_hbm.at[rows, tail_cols], s1, m1),
            pltpu.make_async_copy(e2_hbm.at[rows, tail_cols], s2, m2),
        ]
        for cp in copies:
            cp.start()
        for cp in copies:
            cp.wait()
        t01_ref[...] = jnp.concatenate([s0[...], s1[...]], axis=1)
        t2_ref[...] = jnp.concatenate([s2[...], s2[...]], axis=1)

    return pl.pallas_call(
        body,
        grid=(v // block_rows,),
        in_specs=[pl.BlockSpec(memory_space=pl.ANY)] * 3,
        out_specs=[pl.BlockSpec((block_rows, 128), lambda i: (i, 0))] * 2,
        out_shape=[jax.ShapeDtypeStruct((v, 128), jnp.float32)] * 2,
        scratch_shapes=[pltpu.VMEM((block_rows, 64), jnp.float32)] * 3
        + [pltpu.SemaphoreType.DMA] * 3,
    )(E0, E1, E2)


def _sc_gather(ids3, E0, E1, E2, T01, T2):
    _, nw, nchunk, chunk = ids3.shape
    bpw = nchunk * chunk
    b = nw * bpw
    wave_rows = _WAVE * chunk
    mesh = plsc.VectorSubcoreMesh(core_axis_name="c", subcore_axis_name="s")

    @functools.partial(
        pl.kernel,
        mesh=mesh,
        out_type=[jax.ShapeDtypeStruct((b, 128), jnp.float32)] * 6,
        scratch_types=[
            pltpu.VMEM((nchunk, chunk), jnp.int32),
            pltpu.VMEM((wave_rows, 128), jnp.float32),
            pltpu.VMEM((wave_rows, 128), jnp.float32),
            pltpu.SemaphoreType.DMA,
        ],
        compiler_params=pltpu.CompilerParams(use_tc_tiling_on_sc=True),
    )
    def k(ids_hbm, e0, e1, e2, t01, t2,
          o0l, o0t, o1l, o1t, o2l, o2t, idx_v, rl_v, rt_v, sem):
        wid = lax.axis_index("s") * 2 + lax.axis_index("c")
        base = wid * bpw
        for t, (e, tt, ol, ot) in enumerate(
                ((e0, t01, o0l, o0t), (e1, t01, o1l, o1t),
                 (e2, t2, o2l, o2t))):
            pltpu.sync_copy(ids_hbm.at[t, wid], idx_v)
            for w in range(nchunk // _WAVE):
                copies = []
                for c in range(_WAVE):
                    i_slice = idx_v.at[w * _WAVE + c]
                    dst = pl.ds(c * chunk, chunk)
                    copies.append(pltpu.async_copy(
                        e.at[i_slice, pl.ds(0, 128)], rl_v.at[dst], sem))
                    copies.append(pltpu.async_copy(
                        tt.at[i_slice], rt_v.at[dst], sem))
                for cp in copies:
                    cp.wait()
                out_slice = pl.ds(base + w * wave_rows, wave_rows)
                pltpu.sync_copy(rl_v, ol.at[out_slice])
                pltpu.sync_copy(rt_v, ot.at[out_slice])

    return k(ids3, E0, E1, E2, T01, T2)


def _tc_fused(cls_token, parts, W1, b1, W2, b2, Wg, bg, block_b=512):
    b, c = cls_token.shape
    d = 192
    t = c + 3 * d

    def body(cls_ref, p0l_ref, p0t_ref, p1l_ref, p1t_ref, p2l_ref, p2t_ref,
             w1_ref, b1_ref, w2_ref, b2_ref, wg_ref, bg_ref, out_ref):
        x = jnp.concatenate([
            cls_ref[...],
            p0l_ref[...], p0t_ref[:, 0:64],
            p1l_ref[...], p1t_ref[:, 64:128],
            p2l_ref[...], p2t_ref[:, 0:64],
        ], axis=1)
        x = x.astype(jnp.bfloat16)
        h = jnp.dot(x, w1_ref[...], preferred_element_type=jnp.float32)
        h = h + b1_ref[...]
        h = 0.5 * h * (1.0 + lax.erf(h * 0.7071067811865476))
        meta = jnp.dot(h.astype(jnp.bfloat16), w2_ref[...],
                       preferred_element_type=jnp.float32)
        meta = meta + b2_ref[...]
        g = jnp.dot(x, wg_ref[...], preferred_element_type=jnp.float32)
        gate = jax.nn.sigmoid(g + bg_ref[...])
        out_ref[...] = meta * gate

    const = lambda i: (0, 0)
    batch = lambda i: (i, 0)
    return pl.pallas_call(
        body,
        grid=(b // block_b,),
        in_specs=[
            pl.BlockSpec((block_b, c), batch),
        ] + [pl.BlockSpec((block_b, 128), batch)] * 6 + [
            pl.BlockSpec((t, c), const),
            pl.BlockSpec((1, c), const),
            pl.BlockSpec((c, c), const),
            pl.BlockSpec((1, c), const),
            pl.BlockSpec((t, c), const),
            pl.BlockSpec((1, c), const),
        ],
        out_specs=pl.BlockSpec((block_b, c), batch),
        out_shape=jax.ShapeDtypeStruct((b, c), jnp.float32),
    )(cls_token, *parts, W1.astype(jnp.bfloat16), b1.reshape(1, c),
      W2.astype(jnp.bfloat16), b2.reshape(1, c), Wg.astype(jnp.bfloat16),
      bg.reshape(1, c))


def kernel(cls_token, meta_ids, E0, E1, E2, W1, b1, W2, b2, Wg, bg):
    b = cls_token.shape[0]
    bpw = b // _NUM_WORKERS
    nchunk = bpw // _CHUNK

    ids3 = meta_ids.astype(jnp.int32).T.reshape(3, _NUM_WORKERS, nchunk,
                                                _CHUNK)
    T01, T2 = _tc_tail_split(E0, E1, E2)
    parts = _sc_gather(ids3, E0, E1, E2, T01, T2)
    return _tc_fused(cls_token, parts, W1, b1, W2, b2, Wg, bg)


# speedup vs baseline: 2.7528x; 1.0598x over previous
import functools

import jax
import jax.numpy as jnp
from jax import lax
from jax.experimental import pallas as pl
from jax.experimental.pallas import tpu as pltpu
from jax.experimental.pallas import tpu_sc as plsc

_NUM_WORKERS = 32
_CHUNK = 128
_WAVE = 2


def _tc_tail_split(E0, E1, E2, block_rows=4000):
    v = E0.shape[0]

    def body(e0_ref, e1_ref, e2_ref, t01_ref, t2_ref):
        t01_ref[...] = jnp.concatenate(
            [e0_ref[:, 128:192], e1_ref[:, 128:192]], axis=1)
        t2_ref[...] = jnp.concatenate(
            [e2_ref[:, 128:192], e2_ref[:, 128:192]], axis=1)

    return pl.pallas_call(
        body,
        grid=(v // block_rows,),
        in_specs=[pl.BlockSpec((block_rows, 192), lambda i: (i, 0))] * 3,
        out_specs=[pl.BlockSpec((block_rows, 128), lambda i: (i, 0))] * 2,
        out_shape=[jax.ShapeDtypeStruct((v, 128), jnp.float32)] * 2,
    )(E0, E1, E2)


def _sc_gather(ids3, E0, E1, E2, T01, T2):
    _, nw, nchunk, chunk = ids3.shape
    bpw = nchunk * chunk
    b = nw * bpw
    wave_rows = _WAVE * chunk
    mesh = plsc.VectorSubcoreMesh(core_axis_name="c", subcore_axis_name="s")

    @functools.partial(
        pl.kernel,
        mesh=mesh,
        out_type=[jax.ShapeDtypeStruct((b, 128), jnp.float32)] * 6,
        scratch_types=[
            pltpu.VMEM((nchunk, chunk), jnp.int32),
            pltpu.VMEM((wave_rows, 128), jnp.float32),
            pltpu.VMEM((wave_rows, 128), jnp.float32),
            pltpu.SemaphoreType.DMA,
        ],
        compiler_params=pltpu.CompilerParams(use_tc_tiling_on_sc=True),
    )
    def k(ids_hbm, e0, e1, e2, t01, t2,
          o0l, o0t, o1l, o1t, o2l, o2t, idx_v, rl_v, rt_v, sem):
        wid = lax.axis_index("s") * 2 + lax.axis_index("c")
        base = wid * bpw
        for t, (e, tt, ol, ot) in enumerate(
                ((e0, t01, o0l, o0t), (e1, t01, o1l, o1t),
                 (e2, t2, o2l, o2t))):
            pltpu.sync_copy(ids_hbm.at[t, wid], idx_v)
            for w in range(nchunk // _WAVE):
                copies = []
                for c in range(_WAVE):
                    i_slice = idx_v.at[w * _WAVE + c]
                    dst = pl.ds(c * chunk, chunk)
                    copies.append(pltpu.async_copy(
                        e.at[i_slice, pl.ds(0, 128)], rl_v.at[dst], sem))
                    copies.append(pltpu.async_copy(
                        tt.at[i_slice], rt_v.at[dst], sem))
                for cp in copies:
                    cp.wait()
                out_slice = pl.ds(base + w * wave_rows, wave_rows)
                pltpu.sync_copy(rl_v, ol.at[out_slice])
                pltpu.sync_copy(rt_v, ot.at[out_slice])

    return k(ids3, E0, E1, E2, T01, T2)


def _tc_fused(cls_token, parts, W1, b1, W2, b2, Wg, bg, block_b=512):
    b, c = cls_token.shape
    d = 192
    t = c + 3 * d

    def body(cls_ref, p0l_ref, p0t_ref, p1l_ref, p1t_ref, p2l_ref, p2t_ref,
             w1_ref, b1_ref, w2_ref, b2_ref, wg_ref, bg_ref, out_ref):
        x = jnp.concatenate([
            cls_ref[...],
            p0l_ref[...], p0t_ref[:, 0:64],
            p1l_ref[...], p1t_ref[:, 64:128],
            p2l_ref[...], p2t_ref[:, 0:64],
        ], axis=1)
        x = x.astype(jnp.bfloat16)
        h = jnp.dot(x, w1_ref[...], preferred_element_type=jnp.float32)
        h = h + b1_ref[...]
        h = 0.5 * h * (1.0 + lax.erf(h * 0.7071067811865476))
        meta = jnp.dot(h.astype(jnp.bfloat16), w2_ref[...],
                       preferred_element_type=jnp.float32)
        meta = meta + b2_ref[...]
        g = jnp.dot(x, wg_ref[...], preferred_element_type=jnp.float32)
        gate = jax.nn.sigmoid(g + bg_ref[...])
        out_ref[...] = meta * gate

    const = lambda i: (0, 0)
    batch = lambda i: (i, 0)
    return pl.pallas_call(
        body,
        grid=(b // block_b,),
        in_specs=[
            pl.BlockSpec((block_b, c), batch),
        ] + [pl.BlockSpec((block_b, 128), batch)] * 6 + [
            pl.BlockSpec((t, c), const),
            pl.BlockSpec((1, c), const),
            pl.BlockSpec((c, c), const),
            pl.BlockSpec((1, c), const),
            pl.BlockSpec((t, c), const),
            pl.BlockSpec((1, c), const),
        ],
        out_specs=pl.BlockSpec((block_b, c), batch),
        out_shape=jax.ShapeDtypeStruct((b, c), jnp.float32),
    )(cls_token, *parts, W1.astype(jnp.bfloat16), b1.reshape(1, c),
      W2.astype(jnp.bfloat16), b2.reshape(1, c), Wg.astype(jnp.bfloat16),
      bg.reshape(1, c))


def kernel(cls_token, meta_ids, E0, E1, E2, W1, b1, W2, b2, Wg, bg):
    b = cls_token.shape[0]
    bpw = b // _NUM_WORKERS
    nchunk = bpw // _CHUNK

    ids3 = meta_ids.astype(jnp.int32).T.reshape(3, _NUM_WORKERS, nchunk,
                                                _CHUNK)
    T01, T2 = _tc_tail_split(E0, E1, E2)
    parts = _sc_gather(ids3, E0, E1, E2, T01, T2)
    return _tc_fused(cls_token, parts, W1, b1, W2, b2, Wg, bg)


# speedup vs baseline: 3.1698x; 1.1515x over previous
import functools

import jax
import jax.numpy as jnp
from jax import lax
from jax.experimental import pallas as pl
from jax.experimental.pallas import tpu as pltpu
from jax.experimental.pallas import tpu_sc as plsc

_NUM_WORKERS = 32
_CHUNK = 128
_WAVE = 2


def _tc_tail_split(E0, E1, E2, block_rows=4000):
    v = E0.shape[0]

    def body(e0_ref, e1_ref, e2_ref, t01_ref, t2_ref):
        t01_ref[...] = jnp.concatenate(
            [e0_ref[:, 0:64], e1_ref[:, 0:64]], axis=1)
        t2_ref[...] = jnp.concatenate(
            [e2_ref[:, 0:64], e2_ref[:, 0:64]], axis=1)

    return pl.pallas_call(
        body,
        grid=(v // block_rows,),
        in_specs=[pl.BlockSpec((block_rows, 128), lambda i: (i, 1))] * 3,
        out_specs=[pl.BlockSpec((block_rows, 128), lambda i: (i, 0))] * 2,
        out_shape=[jax.ShapeDtypeStruct((v, 128), jnp.float32)] * 2,
    )(E0, E1, E2)


def _sc_gather(ids3, E0, E1, E2, T01, T2):
    _, nw, nchunk, chunk = ids3.shape
    bpw = nchunk * chunk
    b = nw * bpw
    wave_rows = _WAVE * chunk
    mesh = plsc.VectorSubcoreMesh(core_axis_name="c", subcore_axis_name="s")

    @functools.partial(
        pl.kernel,
        mesh=mesh,
        out_type=[jax.ShapeDtypeStruct((b, 128), jnp.float32)] * 6,
        scratch_types=[
            pltpu.VMEM((nchunk, chunk), jnp.int32),
            pltpu.VMEM((wave_rows, 128), jnp.float32),
            pltpu.VMEM((wave_rows, 128), jnp.float32),
            pltpu.SemaphoreType.DMA,
        ],
        compiler_params=pltpu.CompilerParams(use_tc_tiling_on_sc=True),
    )
    def k(ids_hbm, e0, e1, e2, t01, t2,
          o0l, o0t, o1l, o1t, o2l, o2t, idx_v, rl_v, rt_v, sem):
        wid = lax.axis_index("s") * 2 + lax.axis_index("c")
        base = wid * bpw
        for t, (e, tt, ol, ot) in enumerate(
                ((e0, t01, o0l, o0t), (e1, t01, o1l, o1t),
                 (e2, t2, o2l, o2t))):
            pltpu.sync_copy(ids_hbm.at[t, wid], idx_v)
            for w in range(nchunk // _WAVE):
                copies = []
                for c in range(_WAVE):
                    i_slice = idx_v.at[w * _WAVE + c]
                    dst = pl.ds(c * chunk, chunk)
                    copies.append(pltpu.async_copy(
                        e.at[i_slice, pl.ds(0, 128)], rl_v.at[dst], sem))
                    copies.append(pltpu.async_copy(
                        tt.at[i_slice], rt_v.at[dst], sem))
                for cp in copies:
                    cp.wait()
                out_slice = pl.ds(base + w * wave_rows, wave_rows)
                pltpu.sync_copy(rl_v, ol.at[out_slice])
                pltpu.sync_copy(rt_v, ot.at[out_slice])

    return k(ids3, E0, E1, E2, T01, T2)


def _tc_fused(cls_token, parts, W1, b1, W2, b2, Wg, bg, block_b=512):
    b, c = cls_token.shape
    d = 192
    t = c + 3 * d

    def body(cls_ref, p0l_ref, p0t_ref, p1l_ref, p1t_ref, p2l_ref, p2t_ref,
             w1g_ref, b1g_ref, w2_ref, b2_ref, out_ref):
        x = jnp.concatenate([
            cls_ref[...],
            p0l_ref[...], p0t_ref[:, 0:64],
            p1l_ref[...], p1t_ref[:, 64:128],
            p2l_ref[...], p2t_ref[:, 0:64],
        ], axis=1)
        x = x.astype(jnp.bfloat16)
        hg = jnp.dot(x, w1g_ref[...], preferred_element_type=jnp.float32)
        hg = hg + b1g_ref[...]
        h = hg[:, 0:c]
        h = 0.5 * h * (1.0 + lax.erf(h * 0.7071067811865476))
        meta = jnp.dot(h.astype(jnp.bfloat16), w2_ref[...],
                       preferred_element_type=jnp.float32)
        meta = meta + b2_ref[...]
        gate = jax.nn.sigmoid(hg[:, c:2 * c])
        out_ref[...] = meta * gate

    const = lambda i: (0, 0)
    batch = lambda i: (i, 0)
    W1g = jnp.concatenate([W1, Wg], axis=1).astype(jnp.bfloat16)
    b1g = jnp.concatenate([b1, bg]).reshape(1, 2 * c)
    return pl.pallas_call(
        body,
        grid=(b // block_b,),
        in_specs=[
            pl.BlockSpec((block_b, c), batch),
        ] + [pl.BlockSpec((block_b, 128), batch)] * 6 + [
            pl.BlockSpec((t, 2 * c), const),
            pl.BlockSpec((1, 2 * c), const),
            pl.BlockSpec((c, c), const),
            pl.BlockSpec((1, c), const),
        ],
        out_specs=pl.BlockSpec((block_b, c), batch),
        out_shape=jax.ShapeDtypeStruct((b, c), jnp.float32),
    )(cls_token, *parts, W1g, b1g,
      W2.astype(jnp.bfloat16), b2.reshape(1, c))


def kernel(cls_token, meta_ids, E0, E1, E2, W1, b1, W2, b2, Wg, bg):
    b = cls_token.shape[0]
    bpw = b // _NUM_WORKERS
    nchunk = bpw // _CHUNK

    ids3 = meta_ids.astype(jnp.int32).T.reshape(3, _NUM_WORKERS, nchunk,
                                                _CHUNK)
    T01, T2 = _tc_tail_split(E0, E1, E2)
    parts = _sc_gather(ids3, E0, E1, E2, T01, T2)
    return _tc_fused(cls_token, parts, W1, b1, W2, b2, Wg, bg)


# speedup vs baseline: 3.1758x; 1.0019x over previous
import functools

import jax
import jax.numpy as jnp
from jax import lax
from jax.experimental import pallas as pl
from jax.experimental.pallas import tpu as pltpu
from jax.experimental.pallas import tpu_sc as plsc

_NUM_WORKERS = 32
_CHUNK = 128
_WAVE = 2


def _tc_tail_split(E0, E1, E2, block_rows=4000):
    v = E0.shape[0]

    def body(e0_ref, e1_ref, e2_ref, t01_ref, t2_ref):
        t01_ref[...] = jnp.concatenate(
            [e0_ref[:, 0:64], e1_ref[:, 0:64]], axis=1)
        t2_ref[...] = jnp.concatenate(
            [e2_ref[:, 0:64], e2_ref[:, 0:64]], axis=1)

    return pl.pallas_call(
        body,
        grid=(v // block_rows,),
        in_specs=[pl.BlockSpec((block_rows, 128), lambda i: (i, 1))] * 3,
        out_specs=[pl.BlockSpec((block_rows, 128), lambda i: (i, 0))] * 2,
        out_shape=[jax.ShapeDtypeStruct((v, 128), jnp.float32)] * 2,
    )(E0, E1, E2)


def _sc_gather3(ids3, S0, S1, S2, full_width):
    _, nw, nchunk, chunk = ids3.shape
    bpw = nchunk * chunk
    b = nw * bpw
    wave_rows = _WAVE * chunk
    mesh = plsc.VectorSubcoreMesh(core_axis_name="c", subcore_axis_name="s")

    @functools.partial(
        pl.kernel,
        mesh=mesh,
        out_type=[jax.ShapeDtypeStruct((b, 128), jnp.float32)] * 3,
        scratch_types=[
            pltpu.VMEM((nchunk, chunk), jnp.int32),
            pltpu.VMEM((wave_rows, 128), jnp.float32),
            pltpu.SemaphoreType.DMA,
        ],
        compiler_params=pltpu.CompilerParams(use_tc_tiling_on_sc=True),
    )
    def k(ids_hbm, s0, s1, s2, o0, o1, o2, idx_v, r_v, sem):
        wid = lax.axis_index("s") * 2 + lax.axis_index("c")
        base = wid * bpw
        for t, (e, o) in enumerate(((s0, o0), (s1, o1), (s2, o2))):
            pltpu.sync_copy(ids_hbm.at[t, wid], idx_v)
            for w in range(nchunk // _WAVE):
                copies = []
                for c in range(_WAVE):
                    i_slice = idx_v.at[w * _WAVE + c]
                    dst = pl.ds(c * chunk, chunk)
                    src = (e.at[i_slice, pl.ds(0, 128)] if full_width
                           else e.at[i_slice])
                    copies.append(pltpu.async_copy(src, r_v.at[dst], sem))
                for cp in copies:
                    cp.wait()
                out_slice = pl.ds(base + w * wave_rows, wave_rows)
                pltpu.sync_copy(r_v, o.at[out_slice])

    return k(ids3, S0, S1, S2)


def _tc_fused(cls_token, parts, W1, b1, W2, b2, Wg, bg, block_b=512):
    b, c = cls_token.shape
    d = 192
    t = c + 3 * d

    def body(cls_ref, p0l_ref, p0t_ref, p1l_ref, p1t_ref, p2l_ref, p2t_ref,
             w1g_ref, b1g_ref, w2_ref, b2_ref, out_ref):
        x = jnp.concatenate([
            cls_ref[...],
            p0l_ref[...], p0t_ref[:, 0:64],
            p1l_ref[...], p1t_ref[:, 64:128],
            p2l_ref[...], p2t_ref[:, 0:64],
        ], axis=1)
        x = x.astype(jnp.bfloat16)
        hg = jnp.dot(x, w1g_ref[...], preferred_element_type=jnp.float32)
        hg = hg + b1g_ref[...]
        h = hg[:, 0:c]
        h = 0.5 * h * (1.0 + lax.erf(h * 0.7071067811865476))
        meta = jnp.dot(h.astype(jnp.bfloat16), w2_ref[...],
                       preferred_element_type=jnp.float32)
        meta = meta + b2_ref[...]
        gate = jax.nn.sigmoid(hg[:, c:2 * c])
        out_ref[...] = meta * gate

    const = lambda i: (0, 0)
    batch = lambda i: (i, 0)
    W1g = jnp.concatenate([W1, Wg], axis=1).astype(jnp.bfloat16)
    b1g = jnp.concatenate([b1, bg]).reshape(1, 2 * c)
    return pl.pallas_call(
        body,
        grid=(b // block_b,),
        in_specs=[
            pl.BlockSpec((block_b, c), batch),
        ] + [pl.BlockSpec((block_b, 128), batch)] * 6 + [
            pl.BlockSpec((t, 2 * c), const),
            pl.BlockSpec((1, 2 * c), const),
            pl.BlockSpec((c, c), const),
            pl.BlockSpec((1, c), const),
        ],
        out_specs=pl.BlockSpec((block_b, c), batch),
        out_shape=jax.ShapeDtypeStruct((b, c), jnp.float32),
    )(cls_token, *parts, W1g, b1g,
      W2.astype(jnp.bfloat16), b2.reshape(1, c))


def kernel(cls_token, meta_ids, E0, E1, E2, W1, b1, W2, b2, Wg, bg):
    b = cls_token.shape[0]
    bpw = b // _NUM_WORKERS
    nchunk = bpw // _CHUNK

    ids3 = meta_ids.astype(jnp.int32).T.reshape(3, _NUM_WORKERS, nchunk,
                                                _CHUNK)
    T01, T2 = _tc_tail_split(E0, E1, E2)
    l0, l1, l2 = _sc_gather3(ids3, E0, E1, E2, full_width=True)
    t0, t1, t2 = _sc_gather3(ids3, T01, T01, T2, full_width=False)
    parts = (l0, t0, l1, t1, l2, t2)
    return _tc_fused(cls_token, parts, W1, b1, W2, b2, Wg, bg)


# speedup vs baseline: 3.2218x; 1.0145x over previous
import functools

import jax
import jax.numpy as jnp
from jax import lax
from jax.experimental import pallas as pl
from jax.experimental.pallas import tpu as pltpu
from jax.experimental.pallas import tpu_sc as plsc

_NUM_WORKERS = 32
_CHUNK = 128
_WAVE = 2


def _tc_tail_split(E0, E1, E2, block_rows=5000):
    v = E0.shape[0]

    def body(e0_ref, e1_ref, e2_ref, t01_ref, t2_ref):
        t01_ref[...] = jnp.concatenate(
            [e0_ref[:, 0:64], e1_ref[:, 0:64]], axis=1)
        t2_ref[...] = jnp.concatenate(
            [e2_ref[:, 0:64], e2_ref[:, 0:64]], axis=1)

    return pl.pallas_call(
        body,
        grid=(v // block_rows,),
        in_specs=[pl.BlockSpec((block_rows, 128), lambda i: (i, 1))] * 3,
        out_specs=[pl.BlockSpec((block_rows, 128), lambda i: (i, 0))] * 2,
        out_shape=[jax.ShapeDtypeStruct((v, 128), jnp.float32)] * 2,
    )(E0, E1, E2)


def _sc_gather3(ids3, S0, S1, S2, full_width):
    _, nw, nchunk, chunk = ids3.shape
    bpw = nchunk * chunk
    b = nw * bpw
    wave_rows = _WAVE * chunk
    mesh = plsc.VectorSubcoreMesh(core_axis_name="c", subcore_axis_name="s")

    @functools.partial(
        pl.kernel,
        mesh=mesh,
        out_type=[jax.ShapeDtypeStruct((b, 128), jnp.float32)] * 3,
        scratch_types=[
            pltpu.VMEM((nchunk, chunk), jnp.int32),
            pltpu.VMEM((wave_rows, 128), jnp.float32),
            pltpu.SemaphoreType.DMA,
        ],
        compiler_params=pltpu.CompilerParams(use_tc_tiling_on_sc=True),
    )
    def k(ids_hbm, s0, s1, s2, o0, o1, o2, idx_v, r_v, sem):
        wid = lax.axis_index("s") * 2 + lax.axis_index("c")
        base = wid * bpw
        for t, (e, o) in enumerate(((s0, o0), (s1, o1), (s2, o2))):
            pltpu.sync_copy(ids_hbm.at[t, wid], idx_v)
            for w in range(nchunk // _WAVE):
                copies = []
                for c in range(_WAVE):
                    i_slice = idx_v.at[w * _WAVE + c]
                    dst = pl.ds(c * chunk, chunk)
                    src = (e.at[i_slice, pl.ds(0, 128)] if full_width
                           else e.at[i_slice])
                    copies.append(pltpu.async_copy(src, r_v.at[dst], sem))
                for cp in copies:
                    cp.wait()
                out_slice = pl.ds(base + w * wave_rows, wave_rows)
                pltpu.sync_copy(r_v, o.at[out_slice])

    return k(ids3, S0, S1, S2)


def _tc_fused(cls_token, parts, W1, b1, W2, b2, Wg, bg, block_b=1024):
    b, c = cls_token.shape
    d = 192
    t = c + 3 * d

    def body(cls_ref, p0l_ref, p0t_ref, p1l_ref, p1t_ref, p2l_ref, p2t_ref,
             w1g_ref, b1g_ref, w2_ref, b2_ref, out_ref):
        x = jnp.concatenate([
            cls_ref[...],
            p0l_ref[...], p0t_ref[:, 0:64],
            p1l_ref[...], p1t_ref[:, 64:128],
            p2l_ref[...], p2t_ref[:, 0:64],
        ], axis=1)
        x = x.astype(jnp.bfloat16)
        hg = jnp.dot(x, w1g_ref[...], preferred_element_type=jnp.float32)
        hg = hg + b1g_ref[...]
        h = hg[:, 0:c]
        h = 0.5 * h * (1.0 + lax.erf(h * 0.7071067811865476))
        meta = jnp.dot(h.astype(jnp.bfloat16), w2_ref[...],
                       preferred_element_type=jnp.float32)
        meta = meta + b2_ref[...]
        gate = jax.nn.sigmoid(hg[:, c:2 * c])
        out_ref[...] = meta * gate

    const = lambda i: (0, 0)
    batch = lambda i: (i, 0)
    W1g = jnp.concatenate([W1, Wg], axis=1).astype(jnp.bfloat16)
    b1g = jnp.concatenate([b1, bg]).reshape(1, 2 * c)
    return pl.pallas_call(
        body,
        grid=(b // block_b,),
        in_specs=[
            pl.BlockSpec((block_b, c), batch),
        ] + [pl.BlockSpec((block_b, 128), batch)] * 6 + [
            pl.BlockSpec((t, 2 * c), const),
            pl.BlockSpec((1, 2 * c), const),
            pl.BlockSpec((c, c), const),
            pl.BlockSpec((1, c), const),
        ],
        out_specs=pl.BlockSpec((block_b, c), batch),
        out_shape=jax.ShapeDtypeStruct((b, c), jnp.float32),
    )(cls_token, *parts, W1g, b1g,
      W2.astype(jnp.bfloat16), b2.reshape(1, c))


def kernel(cls_token, meta_ids, E0, E1, E2, W1, b1, W2, b2, Wg, bg):
    b = cls_token.shape[0]
    bpw = b // _NUM_WORKERS
    nchunk = bpw // _CHUNK

    ids3 = meta_ids.astype(jnp.int32).T.reshape(3, _NUM_WORKERS, nchunk,
                                                _CHUNK)
    T01, T2 = _tc_tail_split(E0, E1, E2)
    l0, l1, l2 = _sc_gather3(ids3, E0, E1, E2, full_width=True)
    t0, t1, t2 = _sc_gather3(ids3, T01, T01, T2, full_width=False)
    parts = (l0, t0, l1, t1, l2, t2)
    return _tc_fused(cls_token, parts, W1, b1, W2, b2, Wg, bg)
